# baseline (speedup 1.0000x reference)
import functools

import jax
import jax.numpy as jnp
from jax import lax
from jax.experimental import pallas as pl
from jax.experimental.pallas import tpu as pltpu
from jax.experimental.pallas import tpu_sc as plsc

D = 64
H = 4
LANES = 16
NC = 2
NS = 16
NW = NC * NS
CHUNK = 128
BPC = CHUNK // H
VBLK = 4096
NBLK = 62
S2 = NBLK * VBLK
VOCAB = 1000000
LAST_BLK = VOCAB // VBLK


def _pack_pair(a, b):
    ua = lax.bitcast_convert_type(a, jnp.int32)
    ub = lax.bitcast_convert_type(b, jnp.int32)
    hi = (ua + 0x8000) & jnp.int32(-65536)
    lo = lax.shift_right_logical(ub + 0x8000, 16)
    return lax.bitcast_convert_type(hi | lo, jnp.float32)


def _tc_pack_body(t0_ref, t1_ref, t2_ref, t3_ref, wt_ref, o_ref):
    dn = (((0,), (0,)), ((), ()))
    wt = wt_ref[...]
    ys = [lax.dot_general(wt, t_ref[...], dn,
                          preferred_element_type=jnp.float32)
          for t_ref in (t0_ref, t1_ref, t2_ref, t3_ref)]
    p01 = jnp.transpose(_pack_pair(ys[0], ys[1]))
    p23 = jnp.transpose(_pack_pair(ys[2], ys[3]))
    o_ref[...] = jnp.concatenate([p01, p23], axis=1)


def _tc_pack(table_t, wt):
    def spec(q):
        return pl.BlockSpec(
            (D, VBLK),
            lambda i, q=q: (0, jnp.minimum(i + q * NBLK, LAST_BLK)))

    return pl.pallas_call(
        _tc_pack_body,
        grid=(NBLK,),
        in_specs=[spec(0), spec(1), spec(2), spec(3),
                  pl.BlockSpec((D, D), lambda i: (0, 0))],
        out_specs=pl.BlockSpec((VBLK, 2 * D), lambda i: (i, 0)),
        out_shape=jax.ShapeDtypeStruct((S2, 2 * D), jnp.float32),
    )(table_t, table_t, table_t, table_t, wt)


def _sc_pool_body(ids_hbm, p_hbm, bias_hbm, out_hbm, idx_v, off_v, mul_v,
                  bias_v, rows_a, rows_b, out_v, sem_a, sem_b):
    c = lax.axis_index("c")
    s = lax.axis_index("s")
    wid = s * NC + c
    nchunk = ids_hbm.shape[1]
    batch_per_w = nchunk * BPC

    pltpu.sync_copy(ids_hbm.at[wid], idx_v)
    pltpu.sync_copy(bias_hbm, bias_v)

    def prep(ci, _):
        def grp(g, __):
            sl = pl.ds(g * LANES, LANES)
            v = idx_v[ci, sl]
            zero = jnp.zeros((LANES,), jnp.int32)
            q = (jnp.where(v >= S2, 1, zero) + jnp.where(v >= 2 * S2, 1, zero)
                 + jnp.where(v >= 3 * S2, 1, zero))
            idx_v[ci, sl] = v - q * S2
            off_v[ci, sl] = jnp.where(q >= 2, D, zero)
            mul_v[ci, sl] = jnp.where((q & 1) > 0, 65536, 1 + zero)
            return __
        lax.fori_loop(0, CHUNK // LANES, grp, 0, unroll=False)
        return _

    lax.fori_loop(0, nchunk, prep, 0, unroll=False)

    bias_regs = [bias_v[pl.ds(cc * LANES, LANES)] for cc in range(D // LANES)]

    def fire(ci, buf, sem):
        pltpu.async_copy(p_hbm.at[idx_v.at[ci]], buf, sem)

    def drain(buf, sem):
        pltpu.make_async_copy(p_hbm.at[pl.ds(0, CHUNK)], buf, sem).wait()

    def pool(ci, buf):
        def grp(g, _):
            off_vec = off_v[ci, pl.ds(g * LANES, LANES)]
            mul_vec = mul_v[ci, pl.ds(g * LANES, LANES)]
            for bb in range(LANES // H):
                b = g * (LANES // H) + bb
                offs = [off_vec[H * bb + j] for j in range(H)]
                muls = [mul_vec[H * bb + j] for j in range(H)]
                for cc in range(D // LANES):
                    v = bias_regs[cc]
                    for j in range(H):
                        w = lax.bitcast_convert_type(
                            buf[g * LANES + H * bb + j,
                                pl.ds(offs[j] + cc * LANES, LANES)], jnp.int32)
                        v = v + lax.bitcast_convert_type(
                            (w * muls[j]) & jnp.int32(-65536), jnp.float32)
                    out_v[ci * BPC + b, pl.ds(cc * LANES, LANES)] = (
                        jnp.maximum(v, 0.0))
            return _

        lax.fori_loop(0, CHUNK // LANES, grp, 0, unroll=False)

    fire(0, rows_a, sem_a)

    def pair_body(i, _):
        ca = 2 * i
        fire(ca + 1, rows_b, sem_b)
        drain(rows_a, sem_a)
        pool(ca, rows_a)

        @pl.when(i < nchunk // 2 - 1)
        def _fire_next():
            fire(ca + 2, rows_a, sem_a)

        drain(rows_b, sem_b)
        pool(ca + 1, rows_b)
        return _

    lax.fori_loop(0, nchunk // 2, pair_body, 0, unroll=False)
    pltpu.sync_copy(out_v, out_hbm.at[pl.ds(wid * batch_per_w, batch_per_w)])


def _sc_pool(ids_r, p_packed, bias):
    nw, nchunk, chunk = ids_r.shape
    batch_per_w = nchunk * BPC
    batch = nw * batch_per_w
    mesh = plsc.VectorSubcoreMesh(core_axis_name="c", subcore_axis_name="s")
    kern = pl.kernel(
        _sc_pool_body,
        mesh=mesh,
        out_type=jax.ShapeDtypeStruct((batch, D), jnp.float32),
        scratch_types=[
            pltpu.VMEM((nchunk, chunk), jnp.int32),
            pltpu.VMEM((nchunk, chunk), jnp.int32),
            pltpu.VMEM((nchunk, chunk), jnp.int32),
            pltpu.VMEM((D,), jnp.float32),
            pltpu.VMEM((chunk, 2 * D), jnp.float32),
            pltpu.VMEM((chunk, 2 * D), jnp.float32),
            pltpu.VMEM((batch_per_w, D), jnp.float32),
            pltpu.SemaphoreType.DMA,
            pltpu.SemaphoreType.DMA,
        ],
    )
    return kern(ids_r, p_packed, bias)


def kernel(ids, emb_table, proj_w, proj_b):
    batch = ids.shape[0]
    ids_per_w = batch * H // NW
    ids_r = ids.reshape(NW, ids_per_w // CHUNK, CHUNK)
    wt = proj_w.T * (1.0 / H)
    p_packed = _tc_pack(emb_table.T, wt)
    return _sc_pool(ids_r, p_packed, proj_b)


# speedup vs baseline: 1.0850x; 1.0850x over previous
import functools

import jax
import jax.numpy as jnp
from jax import lax
from jax.experimental import pallas as pl
from jax.experimental.pallas import tpu as pltpu
from jax.experimental.pallas import tpu_sc as plsc

D = 64
H = 4
LANES = 16
NC = 2
NS = 16
NW = NC * NS
CHUNK = 128
BPC = CHUNK // H
VBLK = 8192
NBLK = 31
S2 = NBLK * VBLK
VOCAB = 1000000
LAST_BLK = VOCAB // VBLK


def _pack_pair(a, b):
    ua = lax.bitcast_convert_type(a, jnp.int32)
    ub = lax.bitcast_convert_type(b, jnp.int32)
    hi = (ua + 0x8000) & jnp.int32(-65536)
    lo = lax.shift_right_logical(ub + 0x8000, 16)
    return lax.bitcast_convert_type(hi | lo, jnp.float32)


def _tc_pack_body(t0_ref, t1_ref, t2_ref, t3_ref, wt_ref, o_ref):
    dn = (((0,), (0,)), ((), ()))
    wt = wt_ref[...]
    ys = [lax.dot_general(wt, t_ref[...], dn,
                          preferred_element_type=jnp.float32)
          for t_ref in (t0_ref, t1_ref, t2_ref, t3_ref)]
    p01 = jnp.transpose(_pack_pair(ys[0], ys[1]))
    p23 = jnp.transpose(_pack_pair(ys[2], ys[3]))
    o_ref[...] = jnp.concatenate([p01, p23], axis=1)


def _tc_pack(table_t, wt):
    def spec(q):
        return pl.BlockSpec(
            (D, VBLK),
            lambda i, q=q: (0, jnp.minimum(i + q * NBLK, LAST_BLK)))

    return pl.pallas_call(
        _tc_pack_body,
        grid=(NBLK,),
        in_specs=[spec(0), spec(1), spec(2), spec(3),
                  pl.BlockSpec((D, D), lambda i: (0, 0))],
        out_specs=pl.BlockSpec((VBLK, 2 * D), lambda i: (i, 0)),
        out_shape=jax.ShapeDtypeStruct((S2, 2 * D), jnp.float32),
    )(table_t, table_t, table_t, table_t, wt)


def _sc_pool_body(ids_hbm, p_hbm, bias_hbm, out_hbm, idx_v, off_v, mul_v,
                  bias_v, rows_a, rows_b, out_v, sem_a, sem_b):
    c = lax.axis_index("c")
    s = lax.axis_index("s")
    wid = s * NC + c
    nchunk = ids_hbm.shape[1]
    batch_per_w = nchunk * BPC

    pltpu.sync_copy(ids_hbm.at[wid], idx_v)
    pltpu.sync_copy(bias_hbm, bias_v)

    def prep(ci, _):
        def grp(g, __):
            sl = pl.ds(g * LANES, LANES)
            v = idx_v[ci, sl]
            zero = jnp.zeros((LANES,), jnp.int32)
            q = (jnp.where(v >= S2, 1, zero) + jnp.where(v >= 2 * S2, 1, zero)
                 + jnp.where(v >= 3 * S2, 1, zero))
            idx_v[ci, sl] = v - q * S2
            off_v[ci, sl] = jnp.where(q >= 2, D, zero)
            mul_v[ci, sl] = jnp.where((q & 1) > 0, 65536, 1 + zero)
            return __
        lax.fori_loop(0, CHUNK // LANES, grp, 0, unroll=False)
        return _

    lax.fori_loop(0, nchunk, prep, 0, unroll=False)

    bias_regs = [bias_v[pl.ds(cc * LANES, LANES)] for cc in range(D // LANES)]

    def fire(ci, buf, sem):
        pltpu.async_copy(p_hbm.at[idx_v.at[ci]], buf, sem)

    def drain(buf, sem):
        pltpu.make_async_copy(p_hbm.at[pl.ds(0, CHUNK)], buf, sem).wait()

    def pool(ci, buf):
        def grp(g, _):
            off_vec = off_v[ci, pl.ds(g * LANES, LANES)]
            mul_vec = mul_v[ci, pl.ds(g * LANES, LANES)]
            for bb in range(LANES // H):
                b = g * (LANES // H) + bb
                offs = [off_vec[H * bb + j] for j in range(H)]
                muls = [mul_vec[H * bb + j] for j in range(H)]
                for cc in range(D // LANES):
                    v = bias_regs[cc]
                    for j in range(H):
                        w = lax.bitcast_convert_type(
                            buf[g * LANES + H * bb + j,
                                pl.ds(offs[j] + cc * LANES, LANES)], jnp.int32)
                        v = v + lax.bitcast_convert_type(
                            (w * muls[j]) & jnp.int32(-65536), jnp.float32)
                    out_v[ci * BPC + b, pl.ds(cc * LANES, LANES)] = (
                        jnp.maximum(v, 0.0))
            return _

        lax.fori_loop(0, CHUNK // LANES, grp, 0, unroll=False)

    fire(0, rows_a, sem_a)

    def pair_body(i, _):
        ca = 2 * i
        fire(ca + 1, rows_b, sem_b)
        drain(rows_a, sem_a)
        pool(ca, rows_a)

        @pl.when(i < nchunk // 2 - 1)
        def _fire_next():
            fire(ca + 2, rows_a, sem_a)

        drain(rows_b, sem_b)
        pool(ca + 1, rows_b)
        return _

    lax.fori_loop(0, nchunk // 2, pair_body, 0, unroll=False)
    pltpu.sync_copy(out_v, out_hbm.at[pl.ds(wid * batch_per_w, batch_per_w)])


def _sc_pool(ids_r, p_packed, bias):
    nw, nchunk, chunk = ids_r.shape
    batch_per_w = nchunk * BPC
    batch = nw * batch_per_w
    mesh = plsc.VectorSubcoreMesh(core_axis_name="c", subcore_axis_name="s")
    kern = pl.kernel(
        _sc_pool_body,
        mesh=mesh,
        out_type=jax.ShapeDtypeStruct((batch, D), jnp.float32),
        scratch_types=[
            pltpu.VMEM((nchunk, chunk), jnp.int32),
            pltpu.VMEM((nchunk, chunk), jnp.int32),
            pltpu.VMEM((nchunk, chunk), jnp.int32),
            pltpu.VMEM((D,), jnp.float32),
            pltpu.VMEM((chunk, 2 * D), jnp.float32),
            pltpu.VMEM((chunk, 2 * D), jnp.float32),
            pltpu.VMEM((batch_per_w, D), jnp.float32),
            pltpu.SemaphoreType.DMA,
            pltpu.SemaphoreType.DMA,
        ],
    )
    return kern(ids_r, p_packed, bias)


def kernel(ids, emb_table, proj_w, proj_b):
    batch = ids.shape[0]
    ids_per_w = batch * H // NW
    ids_r = ids.reshape(NW, ids_per_w // CHUNK, CHUNK)
    wt = proj_w.T * (1.0 / H)
    p_packed = _tc_pack(emb_table.T, wt)
    return _sc_pool(ids_r, p_packed, proj_b)


# speedup vs baseline: 1.0868x; 1.0017x over previous
import functools

import jax
import jax.numpy as jnp
from jax import lax
from jax.experimental import pallas as pl
from jax.experimental.pallas import tpu as pltpu
from jax.experimental.pallas import tpu_sc as plsc

D = 64
H = 4
LANES = 16
NC = 2
NS = 16
NW = NC * NS
CHUNK = 128
BPC = CHUNK // H
VBLK = 8192
NBLK = 31
S2 = NBLK * VBLK
VOCAB = 1000000
LAST_BLK = VOCAB // VBLK


def _pack_pair(a, b):
    ua = lax.bitcast_convert_type(a, jnp.int32)
    ub = lax.bitcast_convert_type(b, jnp.int32)
    hi = (ua + 0x8000) & jnp.int32(-65536)
    lo = lax.shift_right_logical(ub + 0x8000, 16)
    return lax.bitcast_convert_type(hi | lo, jnp.float32)


def _tc_pack_body(t0_ref, t1_ref, t2_ref, t3_ref, wt_ref, o_ref):
    dn = (((0,), (0,)), ((), ()))
    wt = wt_ref[...]
    ys = [lax.dot_general(wt, t_ref[...], dn,
                          preferred_element_type=jnp.float32)
          for t_ref in (t0_ref, t1_ref, t2_ref, t3_ref)]
    p01 = jnp.transpose(_pack_pair(ys[0], ys[1]))
    p23 = jnp.transpose(_pack_pair(ys[2], ys[3]))
    o_ref[...] = jnp.concatenate([p01, p23], axis=1)


def _tc_pack(table_t, wt):
    def spec(q):
        return pl.BlockSpec(
            (D, VBLK),
            lambda i, q=q: (0, jnp.minimum(i + q * NBLK, LAST_BLK)))

    return pl.pallas_call(
        _tc_pack_body,
        grid=(NBLK,),
        in_specs=[spec(0), spec(1), spec(2), spec(3),
                  pl.BlockSpec((D, D), lambda i: (0, 0))],
        out_specs=pl.BlockSpec((VBLK, 2 * D), lambda i: (i, 0)),
        out_shape=jax.ShapeDtypeStruct((S2, 2 * D), jnp.float32),
    )(table_t, table_t, table_t, table_t, wt)


def _sc_pool_body(ids_hbm, p_hbm, bias_hbm, out_hbm, idx_v, off_v, mul_v,
                  bias_v, rows_a, rows_b, out_v, sem_a, sem_b):
    c = lax.axis_index("c")
    s = lax.axis_index("s")
    wid = s * NC + c
    nchunk = ids_hbm.shape[1]
    batch_per_w = nchunk * BPC

    pltpu.sync_copy(ids_hbm.at[wid], idx_v)
    pltpu.sync_copy(bias_hbm, bias_v)

    def prep(ci, _):
        def grp(g, __):
            sl = pl.ds(g * LANES, LANES)
            v = idx_v[ci, sl]
            zero = jnp.zeros((LANES,), jnp.int32)
            q = (jnp.where(v >= S2, 1, zero) + jnp.where(v >= 2 * S2, 1, zero)
                 + jnp.where(v >= 3 * S2, 1, zero))
            idx_v[ci, sl] = v - q * S2
            off_v[ci, sl] = (jnp.where(q >= 2, D, zero)
                             + jnp.where((q & 1) > 0, 65536, 1 + zero))
            return __
        lax.fori_loop(0, CHUNK // LANES, grp, 0, unroll=False)
        return _

    lax.fori_loop(0, nchunk, prep, 0, unroll=False)

    bias_regs = [bias_v[pl.ds(cc * LANES, LANES)] for cc in range(D // LANES)]

    def fire(ci, buf, sem):
        pltpu.async_copy(p_hbm.at[idx_v.at[ci]], buf, sem)

    def drain(buf, sem):
        pltpu.make_async_copy(p_hbm.at[pl.ds(0, CHUNK)], buf, sem).wait()

    def pool(ci, buf):
        def grp(g, _):
            off_vec = off_v[ci, pl.ds(g * LANES, LANES)]
            for bb in range(LANES // H):
                b = g * (LANES // H) + bb
                codes = [off_vec[H * bb + j] for j in range(H)]
                offs = [cd & D for cd in codes]
                muls = [cd & 65537 for cd in codes]
                for cc in range(D // LANES):
                    v = bias_regs[cc]
                    for j in range(H):
                        w = lax.bitcast_convert_type(
                            buf[g * LANES + H * bb + j,
                                pl.ds(offs[j] + cc * LANES, LANES)], jnp.int32)
                        v = v + lax.bitcast_convert_type(
                            (w * muls[j]) & jnp.int32(-65536), jnp.float32)
                    out_v[ci * BPC + b, pl.ds(cc * LANES, LANES)] = (
                        jnp.maximum(v, 0.0))
            return _

        lax.fori_loop(0, CHUNK // LANES, grp, 0, unroll=False)

    fire(0, rows_a, sem_a)

    def pair_body(i, _):
        ca = 2 * i
        fire(ca + 1, rows_b, sem_b)
        drain(rows_a, sem_a)
        pool(ca, rows_a)

        @pl.when(i < nchunk // 2 - 1)
        def _fire_next():
            fire(ca + 2, rows_a, sem_a)

        drain(rows_b, sem_b)
        pool(ca + 1, rows_b)
        return _

    lax.fori_loop(0, nchunk // 2, pair_body, 0, unroll=False)
    pltpu.sync_copy(out_v, out_hbm.at[pl.ds(wid * batch_per_w, batch_per_w)])


def _sc_pool(ids_r, p_packed, bias):
    nw, nchunk, chunk = ids_r.shape
    batch_per_w = nchunk * BPC
    batch = nw * batch_per_w
    mesh = plsc.VectorSubcoreMesh(core_axis_name="c", subcore_axis_name="s")
    kern = pl.kernel(
        _sc_pool_body,
        mesh=mesh,
        out_type=jax.ShapeDtypeStruct((batch, D), jnp.float32),
        scratch_types=[
            pltpu.VMEM((nchunk, chunk), jnp.int32),
            pltpu.VMEM((nchunk, chunk), jnp.int32),
            pltpu.VMEM((nchunk, chunk), jnp.int32),
            pltpu.VMEM((D,), jnp.float32),
            pltpu.VMEM((chunk, 2 * D), jnp.float32),
            pltpu.VMEM((chunk, 2 * D), jnp.float32),
            pltpu.VMEM((batch_per_w, D), jnp.float32),
            pltpu.SemaphoreType.DMA,
            pltpu.SemaphoreType.DMA,
        ],
    )
    return kern(ids_r, p_packed, bias)


def kernel(ids, emb_table, proj_w, proj_b):
    batch = ids.shape[0]
    ids_per_w = batch * H // NW
    ids_r = ids.reshape(NW, ids_per_w // CHUNK, CHUNK)
    wt = proj_w.T * (1.0 / H)
    p_packed = _tc_pack(emb_table.T, wt)
    return _sc_pool(ids_r, p_packed, proj_b)
